# baseline (speedup 1.0000x reference)
import jax
import jax.numpy as jnp
import numpy as np
from jax import lax
from jax.experimental import pallas as pl
from jax.experimental.pallas import tpu as pltpu
from jax.experimental.pallas import tpu_sc as plsc

_B, _N, _C, _H, _K = 1, 2048, 768, 12, 35
_D = _C // _H
_KP = 40
_G = _KP // 8
_CHALF = _C // 2
_SCALE = 1.0 / np.sqrt(_D)

_PROJ_NB = 256
_ATTN_NB = 64

_SC_NC = 2
_SC_NS = 16
_NW = _SC_NC * _SC_NS
_MSC = 1024
_MTC = _N - _MSC
_NIDX = _MSC * _KP
_CH = 128


def _pack_bf16_pair(hi_f32, lo_f32):
    hi = lax.bitcast_convert_type(hi_f32, jnp.int32)
    lo = lax.bitcast_convert_type(lo_f32, jnp.int32)
    hi = (hi + 0x8000) & jnp.int32(-65536)
    lo = ((lo + 0x8000) >> 16) & jnp.int32(0xFFFF)
    return hi | lo


def _unpack_hi(packed):
    return lax.bitcast_convert_type(packed & jnp.int32(-65536), jnp.float32)


def _unpack_lo(packed):
    return lax.bitcast_convert_type(packed << 16, jnp.float32)


def _proj_body(xq_ref, xk_ref, xv_ref, wq_ref, bq_ref, wq1_ref, bq1_ref,
               wk_ref, bk_ref, wv_ref, bv_ref,
               q_ref, q1_ref, kbf_ref, vbf_ref, vp_ref):
    xq = xq_ref[...]
    q_ref[...] = jnp.dot(xq, wq_ref[...], preferred_element_type=jnp.float32) + bq_ref[...]
    q1_ref[...] = jnp.dot(xq, wq1_ref[...], preferred_element_type=jnp.float32) + bq1_ref[...]
    k = jnp.dot(xk_ref[...], wk_ref[...], preferred_element_type=jnp.float32) + bk_ref[...]
    v = jnp.dot(xv_ref[...], wv_ref[...], preferred_element_type=jnp.float32) + bv_ref[...]
    kbf_ref[...] = k.astype(jnp.bfloat16)
    vbf_ref[...] = v.astype(jnp.bfloat16)
    vp_ref[...] = _pack_bf16_pair(v[:, :_CHALF], v[:, _CHALF:])


def _sc_gather_body(vp_hbm, idx_hbm, out_hbm):
    def body(i_vmem, o_vmem):
        pltpu.sync_copy(vp_hbm.at[i_vmem.at[0]], o_vmem)

    pltpu.emit_pipeline(
        body,
        grid=(_NIDX // _CH,),
        in_specs=[pl.BlockSpec((1, _CH), lambda i: (0, i))],
        out_specs=[pl.BlockSpec((_CH, _CHALF), lambda i: (i, 0))],
        core_axis_name=("c", "s"),
        dimension_semantics=(pltpu.PARALLEL,),
    )(idx_hbm, out_hbm)


def _score_body(q_ref, q1_ref, kall_ref, idx_ref, rpe_ref,
                wp_ref, bp_ref, pool_ref, probs_ref):
    nb = q_ref.shape[0]
    q = q_ref[...]
    q1 = q1_ref[...]
    kall = kall_ref[...]
    wp = wp_ref[...]
    bp = bp_ref[...]
    pool = pool_ref[...]
    iota3 = jax.lax.broadcasted_iota(jnp.int32, (nb, 8, _N), 2)

    def rep8(x):
        return jnp.broadcast_to(x[:, None, :], (nb, 8, _C)).reshape(nb * 8, _C)

    q_rep = rep8(q)
    q1_rep = rep8(q1)

    score_groups = []
    for g in range(_G):
        idx_g = idx_ref[:, g * 8:(g + 1) * 8][:, :, None]
        oh = (iota3 == idx_g).astype(jnp.bfloat16).reshape(nb * 8, _N)
        knb = jnp.dot(oh, kall, preferred_element_type=jnp.float32)
        if g * 8 < _K:
            w = min(8, _K - g * 8)
            rpe_g = rpe_ref[:, g * 8:g * 8 + w, :]
            if w < 8:
                rpe_g = jnp.concatenate(
                    [rpe_g, jnp.zeros((nb, 8 - w, _C), jnp.float32)], axis=1)
            rpe_g = rpe_g.reshape(nb * 8, _C).astype(jnp.bfloat16)
            p_g = jnp.dot(rpe_g, wp, preferred_element_type=jnp.float32) + bp
            ep = jnp.dot(q1_rep * p_g, pool, preferred_element_type=jnp.float32)
        else:
            ep = jnp.zeros((nb * 8, _H), jnp.float32)
        e = jnp.dot(q_rep * knb, pool, preferred_element_type=jnp.float32)
        score_groups.append(((e + ep) * _SCALE).reshape(nb, 8, _H))

    s = jnp.concatenate(score_groups, axis=1)
    kpos = jax.lax.broadcasted_iota(jnp.int32, (nb, _KP, _H), 1)
    s = jnp.where(kpos < _K, s, -1e30)
    m = jnp.max(s, axis=1, keepdims=True)
    ex = jnp.exp(s - m)
    probs_ref[...] = ex / jnp.sum(ex, axis=1, keepdims=True)


def _final_tc_body(vall_ref, idx_ref, pf_ref, poolt_ref, hid_ref, probs_ref):
    nb = pf_ref.shape[0]
    pf = pf_ref[...]
    probs_ref[...] = pf[:, :_K, :]
    vall = vall_ref[...]
    poolt = poolt_ref[...]
    iota3 = jax.lax.broadcasted_iota(jnp.int32, (nb, 8, _N), 2)

    acc = jnp.zeros((nb, _C), dtype=jnp.float32)
    for g in range(_G):
        idx_g = idx_ref[:, g * 8:(g + 1) * 8][:, :, None]
        oh = (iota3 == idx_g).astype(jnp.bfloat16).reshape(nb * 8, _N)
        vnb = jnp.dot(oh, vall, preferred_element_type=jnp.float32)
        pr_g = pf[:, g * 8:(g + 1) * 8, :].reshape(nb * 8, _H)
        w = jnp.dot(pr_g, poolt, preferred_element_type=jnp.float32)
        acc = acc + jnp.sum((w * vnb).reshape(nb, 8, _C), axis=1)
    hid_ref[...] = acc


def _final_body(vnb_ref, pf_ref, ph_ref, plo_ref, hid_ref, probs_ref):
    nb = pf_ref.shape[0]
    pf = pf_ref[...]
    probs_ref[...] = pf[:, :_K, :]
    ph = ph_ref[...]
    plo = plo_ref[...]

    acc_hi = jnp.zeros((nb, _CHALF), dtype=jnp.float32)
    acc_lo = jnp.zeros((nb, _CHALF), dtype=jnp.float32)
    for g in range(_G):
        vnb_g = vnb_ref[:, g * 8:(g + 1) * 8, :].reshape(nb * 8, _CHALF)
        pr_g = pf[:, g * 8:(g + 1) * 8, :].reshape(nb * 8, _H)
        w_hi = jnp.dot(pr_g, ph, preferred_element_type=jnp.float32)
        w_lo = jnp.dot(pr_g, plo, preferred_element_type=jnp.float32)
        acc_hi = acc_hi + jnp.sum(
            (w_hi * _unpack_hi(vnb_g)).reshape(nb, 8, _CHALF), axis=1)
        acc_lo = acc_lo + jnp.sum(
            (w_lo * _unpack_lo(vnb_g)).reshape(nb, 8, _CHALF), axis=1)
    hid_ref[...] = jnp.concatenate([acc_hi, acc_lo], axis=1)


def kernel(input_q, input_k, input_v, rpe_knn_embeddings, knn_idx,
           Wq, bq, Wq1, bq1, Wk, bk, Wv, bv, Wp, bp):
    xq = input_q.reshape(_N, _C)
    xk = input_k.reshape(_N, _C)
    xv = input_v.reshape(_N, _C)
    rpe = rpe_knn_embeddings.reshape(_N, _K, _C)
    idx_pad = jnp.pad(knn_idx.reshape(_N, _K).astype(jnp.int32),
                      ((0, 0), (0, _KP - _K)))
    idx_flat = idx_pad[:_MSC].reshape(1, _NIDX)

    b2 = lambda b: b.reshape(1, _C)

    q, q1, k_bf, v_bf, v_pack = pl.pallas_call(
        _proj_body,
        grid=(_N // _PROJ_NB,),
        in_specs=[
            pl.BlockSpec((_PROJ_NB, _C), lambda i: (i, 0)),
            pl.BlockSpec((_PROJ_NB, _C), lambda i: (i, 0)),
            pl.BlockSpec((_PROJ_NB, _C), lambda i: (i, 0)),
        ] + [
            spec for _ in range(4) for spec in (
                pl.BlockSpec((_C, _C), lambda i: (0, 0)),
                pl.BlockSpec((1, _C), lambda i: (0, 0)),
            )
        ],
        out_specs=[pl.BlockSpec((_PROJ_NB, _C), lambda i: (i, 0))] * 4
        + [pl.BlockSpec((_PROJ_NB, _CHALF), lambda i: (i, 0))],
        out_shape=[jax.ShapeDtypeStruct((_N, _C), jnp.float32)] * 2
        + [jax.ShapeDtypeStruct((_N, _C), jnp.bfloat16)] * 2
        + [jax.ShapeDtypeStruct((_N, _CHALF), jnp.int32)],
    )(xq, xk, xv, Wq.T, b2(bq), Wq1.T, b2(bq1), Wk.T, b2(bk), Wv.T, b2(bv))

    mesh = plsc.VectorSubcoreMesh(core_axis_name="c", subcore_axis_name="s")
    sc_gather = pl.kernel(
        _sc_gather_body,
        mesh=mesh,
        out_type=jax.ShapeDtypeStruct((_NIDX, _CHALF), jnp.int32),
        scratch_types=[],
    )
    vnb = sc_gather(v_pack, idx_flat).reshape(_MSC, _KP, _CHALF)

    pool = jnp.repeat(jnp.eye(_H, dtype=jnp.float32), _D, axis=0)
    eye_h = jnp.eye(_H, dtype=jnp.float32)
    pool_hi = jnp.repeat(eye_h[:, :_H // 2], _D, axis=1)
    pool_lo = jnp.repeat(eye_h[:, _H // 2:], _D, axis=1)

    probs_full = pl.pallas_call(
        _score_body,
        grid=(_N // _ATTN_NB,),
        in_specs=[
            pl.BlockSpec((_ATTN_NB, _C), lambda i: (i, 0)),
            pl.BlockSpec((_ATTN_NB, _C), lambda i: (i, 0)),
            pl.BlockSpec((_N, _C), lambda i: (0, 0)),
            pl.BlockSpec((_ATTN_NB, _KP), lambda i: (i, 0)),
            pl.BlockSpec((_ATTN_NB, _K, _C), lambda i: (i, 0, 0)),
            pl.BlockSpec((_C, _C), lambda i: (0, 0)),
            pl.BlockSpec((1, _C), lambda i: (0, 0)),
            pl.BlockSpec((_C, _H), lambda i: (0, 0)),
        ],
        out_specs=pl.BlockSpec((_ATTN_NB, _KP, _H), lambda i: (i, 0, 0)),
        out_shape=jax.ShapeDtypeStruct((_N, _KP, _H), jnp.float32),
    )(q, q1, k_bf, idx_pad, rpe, Wp.T.astype(jnp.bfloat16), b2(bp), pool)

    hid_a, probs_a = pl.pallas_call(
        _final_body,
        grid=(_MSC // _ATTN_NB,),
        in_specs=[
            pl.BlockSpec((_ATTN_NB, _KP, _CHALF), lambda i: (i, 0, 0)),
            pl.BlockSpec((_ATTN_NB, _KP, _H), lambda i: (i, 0, 0)),
            pl.BlockSpec((_H, _CHALF), lambda i: (0, 0)),
            pl.BlockSpec((_H, _CHALF), lambda i: (0, 0)),
        ],
        out_specs=[
            pl.BlockSpec((_ATTN_NB, _C), lambda i: (i, 0)),
            pl.BlockSpec((_ATTN_NB, _K, _H), lambda i: (i, 0, 0)),
        ],
        out_shape=[
            jax.ShapeDtypeStruct((_MSC, _C), jnp.float32),
            jax.ShapeDtypeStruct((_MSC, _K, _H), jnp.float32),
        ],
    )(vnb, probs_full[:_MSC], pool_hi, pool_lo)

    hid_b, probs_b = pl.pallas_call(
        _final_tc_body,
        grid=(_MTC // _ATTN_NB,),
        in_specs=[
            pl.BlockSpec((_N, _C), lambda i: (0, 0)),
            pl.BlockSpec((_ATTN_NB, _KP), lambda i: (i, 0)),
            pl.BlockSpec((_ATTN_NB, _KP, _H), lambda i: (i, 0, 0)),
            pl.BlockSpec((_H, _C), lambda i: (0, 0)),
        ],
        out_specs=[
            pl.BlockSpec((_ATTN_NB, _C), lambda i: (i, 0)),
            pl.BlockSpec((_ATTN_NB, _K, _H), lambda i: (i, 0, 0)),
        ],
        out_shape=[
            jax.ShapeDtypeStruct((_MTC, _C), jnp.float32),
            jax.ShapeDtypeStruct((_MTC, _K, _H), jnp.float32),
        ],
    )(v_bf, idx_pad[_MSC:], probs_full[_MSC:], pool.T)

    hidden = jnp.concatenate([hid_a, hid_b], axis=0).reshape(_B, _N, _C)
    probs_raw = jnp.concatenate([probs_a, probs_b], axis=0)
    attention_probs = probs_raw.transpose(0, 2, 1).reshape(_B, _N, _H, _K)
    return (hidden, attention_probs)


# speedup vs baseline: 1.0104x; 1.0104x over previous
import jax
import jax.numpy as jnp
import numpy as np
from jax import lax
from jax.experimental import pallas as pl
from jax.experimental.pallas import tpu as pltpu
from jax.experimental.pallas import tpu_sc as plsc

_B, _N, _C, _H, _K = 1, 2048, 768, 12, 35
_D = _C // _H
_KP = 40
_G = _KP // 8
_CHALF = _C // 2
_SCALE = 1.0 / np.sqrt(_D)

_PROJ_NB = 256
_ATTN_NB = 64

_SC_NC = 2
_SC_NS = 16
_NW = _SC_NC * _SC_NS
_MSC = 1408
_MTC = _N - _MSC
_NIDX = _MSC * _KP
_CH = 128


def _pack_bf16_pair(hi_f32, lo_f32):
    hi = lax.bitcast_convert_type(hi_f32, jnp.int32)
    lo = lax.bitcast_convert_type(lo_f32, jnp.int32)
    hi = (hi + 0x8000) & jnp.int32(-65536)
    lo = ((lo + 0x8000) >> 16) & jnp.int32(0xFFFF)
    return hi | lo


def _unpack_hi(packed):
    return lax.bitcast_convert_type(packed & jnp.int32(-65536), jnp.float32)


def _unpack_lo(packed):
    return lax.bitcast_convert_type(packed << 16, jnp.float32)


def _proj_body(xq_ref, xk_ref, xv_ref, wq_ref, bq_ref, wq1_ref, bq1_ref,
               wk_ref, bk_ref, wv_ref, bv_ref,
               q_ref, q1_ref, kbf_ref, vbf_ref, vp_ref):
    xq = xq_ref[...]
    q_ref[...] = jnp.dot(xq, wq_ref[...], preferred_element_type=jnp.float32) + bq_ref[...]
    q1_ref[...] = jnp.dot(xq, wq1_ref[...], preferred_element_type=jnp.float32) + bq1_ref[...]
    k = jnp.dot(xk_ref[...], wk_ref[...], preferred_element_type=jnp.float32) + bk_ref[...]
    v = jnp.dot(xv_ref[...], wv_ref[...], preferred_element_type=jnp.float32) + bv_ref[...]
    kbf_ref[...] = k.astype(jnp.bfloat16)
    vbf_ref[...] = v.astype(jnp.bfloat16)
    vp_ref[...] = _pack_bf16_pair(v[:, :_CHALF], v[:, _CHALF:])


def _sc_gather_body(vp_hbm, idx_hbm, out_hbm):
    def body(i_vmem, o_vmem):
        pltpu.sync_copy(vp_hbm.at[i_vmem.at[0]], o_vmem)

    pltpu.emit_pipeline(
        body,
        grid=(_NIDX // _CH,),
        in_specs=[pl.BlockSpec((1, _CH), lambda i: (0, i))],
        out_specs=[pl.BlockSpec((_CH, _CHALF), lambda i: (i, 0))],
        core_axis_name=("c", "s"),
        dimension_semantics=(pltpu.PARALLEL,),
    )(idx_hbm, out_hbm)


def _score_body(q_ref, q1_ref, kall_ref, idx_ref, rpe_ref,
                wp_ref, bp_ref, pool_ref, probs_ref):
    nb = q_ref.shape[0]
    q = q_ref[...]
    q1 = q1_ref[...]
    kall = kall_ref[...]
    wp = wp_ref[...]
    bp = bp_ref[...]
    pool = pool_ref[...]
    iota3 = jax.lax.broadcasted_iota(jnp.int32, (nb, 8, _N), 2)

    def rep8(x):
        return jnp.broadcast_to(x[:, None, :], (nb, 8, _C)).reshape(nb * 8, _C)

    q_rep = rep8(q)
    q1_rep = rep8(q1)

    score_groups = []
    for g in range(_G):
        idx_g = idx_ref[:, g * 8:(g + 1) * 8][:, :, None]
        oh = (iota3 == idx_g).astype(jnp.bfloat16).reshape(nb * 8, _N)
        knb = jnp.dot(oh, kall, preferred_element_type=jnp.float32)
        if g * 8 < _K:
            w = min(8, _K - g * 8)
            rpe_g = rpe_ref[:, g * 8:g * 8 + w, :]
            if w < 8:
                rpe_g = jnp.concatenate(
                    [rpe_g, jnp.zeros((nb, 8 - w, _C), jnp.float32)], axis=1)
            rpe_g = rpe_g.reshape(nb * 8, _C).astype(jnp.bfloat16)
            p_g = jnp.dot(rpe_g, wp, preferred_element_type=jnp.float32) + bp
            ep = jnp.dot(q1_rep * p_g, pool, preferred_element_type=jnp.float32)
        else:
            ep = jnp.zeros((nb * 8, _H), jnp.float32)
        e = jnp.dot(q_rep * knb, pool, preferred_element_type=jnp.float32)
        score_groups.append(((e + ep) * _SCALE).reshape(nb, 8, _H))

    s = jnp.concatenate(score_groups, axis=1)
    kpos = jax.lax.broadcasted_iota(jnp.int32, (nb, _KP, _H), 1)
    s = jnp.where(kpos < _K, s, -1e30)
    m = jnp.max(s, axis=1, keepdims=True)
    ex = jnp.exp(s - m)
    probs_ref[...] = ex / jnp.sum(ex, axis=1, keepdims=True)


def _final_tc_body(vall_ref, idx_ref, pf_ref, poolt_ref, hid_ref, probs_ref):
    nb = pf_ref.shape[0]
    pf = pf_ref[...]
    probs_ref[...] = pf[:, :_K, :]
    vall = vall_ref[...]
    poolt = poolt_ref[...]
    iota3 = jax.lax.broadcasted_iota(jnp.int32, (nb, 8, _N), 2)

    acc = jnp.zeros((nb, _C), dtype=jnp.float32)
    for g in range(_G):
        idx_g = idx_ref[:, g * 8:(g + 1) * 8][:, :, None]
        oh = (iota3 == idx_g).astype(jnp.bfloat16).reshape(nb * 8, _N)
        vnb = jnp.dot(oh, vall, preferred_element_type=jnp.float32)
        pr_g = pf[:, g * 8:(g + 1) * 8, :].reshape(nb * 8, _H)
        w = jnp.dot(pr_g, poolt, preferred_element_type=jnp.float32)
        acc = acc + jnp.sum((w * vnb).reshape(nb, 8, _C), axis=1)
    hid_ref[...] = acc


def _final_body(vnb_ref, pf_ref, ph_ref, plo_ref, hid_ref, probs_ref):
    nb = pf_ref.shape[0]
    pf = pf_ref[...]
    probs_ref[...] = pf[:, :_K, :]
    ph = ph_ref[...]
    plo = plo_ref[...]

    acc_hi = jnp.zeros((nb, _CHALF), dtype=jnp.float32)
    acc_lo = jnp.zeros((nb, _CHALF), dtype=jnp.float32)
    for g in range(_G):
        vnb_g = vnb_ref[:, g * 8:(g + 1) * 8, :].reshape(nb * 8, _CHALF)
        pr_g = pf[:, g * 8:(g + 1) * 8, :].reshape(nb * 8, _H)
        w_hi = jnp.dot(pr_g, ph, preferred_element_type=jnp.float32)
        w_lo = jnp.dot(pr_g, plo, preferred_element_type=jnp.float32)
        acc_hi = acc_hi + jnp.sum(
            (w_hi * _unpack_hi(vnb_g)).reshape(nb, 8, _CHALF), axis=1)
        acc_lo = acc_lo + jnp.sum(
            (w_lo * _unpack_lo(vnb_g)).reshape(nb, 8, _CHALF), axis=1)
    hid_ref[...] = jnp.concatenate([acc_hi, acc_lo], axis=1)


def kernel(input_q, input_k, input_v, rpe_knn_embeddings, knn_idx,
           Wq, bq, Wq1, bq1, Wk, bk, Wv, bv, Wp, bp):
    xq = input_q.reshape(_N, _C)
    xk = input_k.reshape(_N, _C)
    xv = input_v.reshape(_N, _C)
    rpe = rpe_knn_embeddings.reshape(_N, _K, _C)
    idx_pad = jnp.pad(knn_idx.reshape(_N, _K).astype(jnp.int32),
                      ((0, 0), (0, _KP - _K)))
    idx_flat = idx_pad[:_MSC].reshape(1, _NIDX)

    b2 = lambda b: b.reshape(1, _C)

    q, q1, k_bf, v_bf, v_pack = pl.pallas_call(
        _proj_body,
        grid=(_N // _PROJ_NB,),
        in_specs=[
            pl.BlockSpec((_PROJ_NB, _C), lambda i: (i, 0)),
            pl.BlockSpec((_PROJ_NB, _C), lambda i: (i, 0)),
            pl.BlockSpec((_PROJ_NB, _C), lambda i: (i, 0)),
        ] + [
            spec for _ in range(4) for spec in (
                pl.BlockSpec((_C, _C), lambda i: (0, 0)),
                pl.BlockSpec((1, _C), lambda i: (0, 0)),
            )
        ],
        out_specs=[pl.BlockSpec((_PROJ_NB, _C), lambda i: (i, 0))] * 4
        + [pl.BlockSpec((_PROJ_NB, _CHALF), lambda i: (i, 0))],
        out_shape=[jax.ShapeDtypeStruct((_N, _C), jnp.float32)] * 2
        + [jax.ShapeDtypeStruct((_N, _C), jnp.bfloat16)] * 2
        + [jax.ShapeDtypeStruct((_N, _CHALF), jnp.int32)],
    )(xq, xk, xv, Wq.T, b2(bq), Wq1.T, b2(bq1), Wk.T, b2(bk), Wv.T, b2(bv))

    mesh = plsc.VectorSubcoreMesh(core_axis_name="c", subcore_axis_name="s")
    sc_gather = pl.kernel(
        _sc_gather_body,
        mesh=mesh,
        out_type=jax.ShapeDtypeStruct((_NIDX, _CHALF), jnp.int32),
        scratch_types=[],
    )
    vnb = sc_gather(v_pack, idx_flat).reshape(_MSC, _KP, _CHALF)

    pool = jnp.repeat(jnp.eye(_H, dtype=jnp.float32), _D, axis=0)
    eye_h = jnp.eye(_H, dtype=jnp.float32)
    pool_hi = jnp.repeat(eye_h[:, :_H // 2], _D, axis=1)
    pool_lo = jnp.repeat(eye_h[:, _H // 2:], _D, axis=1)

    probs_full = pl.pallas_call(
        _score_body,
        grid=(_N // _ATTN_NB,),
        in_specs=[
            pl.BlockSpec((_ATTN_NB, _C), lambda i: (i, 0)),
            pl.BlockSpec((_ATTN_NB, _C), lambda i: (i, 0)),
            pl.BlockSpec((_N, _C), lambda i: (0, 0)),
            pl.BlockSpec((_ATTN_NB, _KP), lambda i: (i, 0)),
            pl.BlockSpec((_ATTN_NB, _K, _C), lambda i: (i, 0, 0)),
            pl.BlockSpec((_C, _C), lambda i: (0, 0)),
            pl.BlockSpec((1, _C), lambda i: (0, 0)),
            pl.BlockSpec((_C, _H), lambda i: (0, 0)),
        ],
        out_specs=pl.BlockSpec((_ATTN_NB, _KP, _H), lambda i: (i, 0, 0)),
        out_shape=jax.ShapeDtypeStruct((_N, _KP, _H), jnp.float32),
    )(q, q1, k_bf, idx_pad, rpe, Wp.T.astype(jnp.bfloat16), b2(bp), pool)

    hid_a, probs_a = pl.pallas_call(
        _final_body,
        grid=(_MSC // _ATTN_NB,),
        in_specs=[
            pl.BlockSpec((_ATTN_NB, _KP, _CHALF), lambda i: (i, 0, 0)),
            pl.BlockSpec((_ATTN_NB, _KP, _H), lambda i: (i, 0, 0)),
            pl.BlockSpec((_H, _CHALF), lambda i: (0, 0)),
            pl.BlockSpec((_H, _CHALF), lambda i: (0, 0)),
        ],
        out_specs=[
            pl.BlockSpec((_ATTN_NB, _C), lambda i: (i, 0)),
            pl.BlockSpec((_ATTN_NB, _K, _H), lambda i: (i, 0, 0)),
        ],
        out_shape=[
            jax.ShapeDtypeStruct((_MSC, _C), jnp.float32),
            jax.ShapeDtypeStruct((_MSC, _K, _H), jnp.float32),
        ],
    )(vnb, probs_full[:_MSC], pool_hi, pool_lo)

    hid_b, probs_b = pl.pallas_call(
        _final_tc_body,
        grid=(_MTC // _ATTN_NB,),
        in_specs=[
            pl.BlockSpec((_N, _C), lambda i: (0, 0)),
            pl.BlockSpec((_ATTN_NB, _KP), lambda i: (i, 0)),
            pl.BlockSpec((_ATTN_NB, _KP, _H), lambda i: (i, 0, 0)),
            pl.BlockSpec((_H, _C), lambda i: (0, 0)),
        ],
        out_specs=[
            pl.BlockSpec((_ATTN_NB, _C), lambda i: (i, 0)),
            pl.BlockSpec((_ATTN_NB, _K, _H), lambda i: (i, 0, 0)),
        ],
        out_shape=[
            jax.ShapeDtypeStruct((_MTC, _C), jnp.float32),
            jax.ShapeDtypeStruct((_MTC, _K, _H), jnp.float32),
        ],
    )(v_bf, idx_pad[_MSC:], probs_full[_MSC:], pool.T)

    hidden = jnp.concatenate([hid_a, hid_b], axis=0).reshape(_B, _N, _C)
    probs_raw = jnp.concatenate([probs_a, probs_b], axis=0)
    attention_probs = probs_raw.transpose(0, 2, 1).reshape(_B, _N, _H, _K)
    return (hidden, attention_probs)


# speedup vs baseline: 1.0114x; 1.0010x over previous
import jax
import jax.numpy as jnp
import numpy as np
from jax import lax
from jax.experimental import pallas as pl
from jax.experimental.pallas import tpu as pltpu
from jax.experimental.pallas import tpu_sc as plsc

_B, _N, _C, _H, _K = 1, 2048, 768, 12, 35
_D = _C // _H
_KP = 40
_G = _KP // 8
_CHALF = _C // 2
_SCALE = 1.0 / np.sqrt(_D)

_PROJ_NB = 256
_ATTN_NB = 64

_SC_NC = 2
_SC_NS = 16
_NW = _SC_NC * _SC_NS
_MSC = 1280
_MTC = _N - _MSC
_NIDX = _MSC * _KP
_CH = 128


def _pack_bf16_pair(hi_f32, lo_f32):
    hi = lax.bitcast_convert_type(hi_f32, jnp.int32)
    lo = lax.bitcast_convert_type(lo_f32, jnp.int32)
    hi = (hi + 0x8000) & jnp.int32(-65536)
    lo = ((lo + 0x8000) >> 16) & jnp.int32(0xFFFF)
    return hi | lo


def _unpack_hi(packed):
    return lax.bitcast_convert_type(packed & jnp.int32(-65536), jnp.float32)


def _unpack_lo(packed):
    return lax.bitcast_convert_type(packed << 16, jnp.float32)


def _proj_body(xq_ref, xk_ref, xv_ref, wq_ref, bq_ref, wq1_ref, bq1_ref,
               wk_ref, bk_ref, wv_ref, bv_ref,
               q_ref, q1_ref, kbf_ref, vbf_ref, vp_ref):
    xq = xq_ref[...]
    q_ref[...] = jnp.dot(xq, wq_ref[...], preferred_element_type=jnp.float32) + bq_ref[...]
    q1_ref[...] = jnp.dot(xq, wq1_ref[...], preferred_element_type=jnp.float32) + bq1_ref[...]
    k = jnp.dot(xk_ref[...], wk_ref[...], preferred_element_type=jnp.float32) + bk_ref[...]
    v = jnp.dot(xv_ref[...], wv_ref[...], preferred_element_type=jnp.float32) + bv_ref[...]
    kbf_ref[...] = k.astype(jnp.bfloat16)
    vbf_ref[...] = v.astype(jnp.bfloat16)
    vp_ref[...] = _pack_bf16_pair(v[:, :_CHALF], v[:, _CHALF:])


def _sc_gather_body(vp_hbm, idx_hbm, out_hbm):
    def body(i_vmem, o_vmem):
        pltpu.sync_copy(vp_hbm.at[i_vmem.at[0]], o_vmem)

    pltpu.emit_pipeline(
        body,
        grid=(_NIDX // _CH,),
        in_specs=[pl.BlockSpec((1, _CH), lambda i: (0, i))],
        out_specs=[pl.BlockSpec((_CH, _CHALF), lambda i: (i, 0))],
        core_axis_name=("c", "s"),
        dimension_semantics=(pltpu.PARALLEL,),
    )(idx_hbm, out_hbm)


def _score_body(q_ref, q1_ref, kall_ref, idx_ref, rpe_ref,
                wp_ref, bp_ref, pool_ref, probs_ref):
    nb = q_ref.shape[0]
    q = q_ref[...]
    q1 = q1_ref[...]
    kall = kall_ref[...]
    wp = wp_ref[...]
    bp = bp_ref[...]
    pool = pool_ref[...]
    iota3 = jax.lax.broadcasted_iota(jnp.int32, (nb, 8, _N), 2)

    def rep8(x):
        return jnp.broadcast_to(x[:, None, :], (nb, 8, _C)).reshape(nb * 8, _C)

    q_rep = rep8(q)
    q1_rep = rep8(q1)

    score_groups = []
    for g in range(_G):
        idx_g = idx_ref[:, g * 8:(g + 1) * 8][:, :, None]
        oh = (iota3 == idx_g).astype(jnp.bfloat16).reshape(nb * 8, _N)
        knb = jnp.dot(oh, kall, preferred_element_type=jnp.float32)
        if g * 8 < _K:
            w = min(8, _K - g * 8)
            rpe_g = rpe_ref[:, g * 8:g * 8 + w, :]
            if w < 8:
                rpe_g = jnp.concatenate(
                    [rpe_g, jnp.zeros((nb, 8 - w, _C), jnp.float32)], axis=1)
            rpe_g = rpe_g.reshape(nb * 8, _C).astype(jnp.bfloat16)
            p_g = jnp.dot(rpe_g, wp, preferred_element_type=jnp.float32) + bp
            ep = jnp.dot(q1_rep * p_g, pool, preferred_element_type=jnp.float32)
        else:
            ep = jnp.zeros((nb * 8, _H), jnp.float32)
        e = jnp.dot(q_rep * knb, pool, preferred_element_type=jnp.float32)
        score_groups.append(((e + ep) * _SCALE).reshape(nb, 8, _H))

    s = jnp.concatenate(score_groups, axis=1)
    kpos = jax.lax.broadcasted_iota(jnp.int32, (nb, _KP, _H), 1)
    s = jnp.where(kpos < _K, s, -1e30)
    m = jnp.max(s, axis=1, keepdims=True)
    ex = jnp.exp(s - m)
    probs_ref[...] = ex / jnp.sum(ex, axis=1, keepdims=True)


def _final_tc_body(vall_ref, idx_ref, pf_ref, poolt_ref, hid_ref, probs_ref):
    nb = pf_ref.shape[0]
    pf = pf_ref[...]
    probs_ref[...] = pf[:, :_K, :]
    vall = vall_ref[...]
    poolt = poolt_ref[...]
    iota3 = jax.lax.broadcasted_iota(jnp.int32, (nb, 8, _N), 2)

    acc = jnp.zeros((nb, _C), dtype=jnp.float32)
    for g in range(_G):
        idx_g = idx_ref[:, g * 8:(g + 1) * 8][:, :, None]
        oh = (iota3 == idx_g).astype(jnp.bfloat16).reshape(nb * 8, _N)
        vnb = jnp.dot(oh, vall, preferred_element_type=jnp.float32)
        pr_g = pf[:, g * 8:(g + 1) * 8, :].reshape(nb * 8, _H)
        w = jnp.dot(pr_g, poolt, preferred_element_type=jnp.float32)
        acc = acc + jnp.sum((w * vnb).reshape(nb, 8, _C), axis=1)
    hid_ref[...] = acc


def _final_body(vnb_ref, pf_ref, ph_ref, plo_ref, hid_ref, probs_ref):
    nb = pf_ref.shape[0]
    pf = pf_ref[...]
    probs_ref[...] = pf[:, :_K, :]
    ph = ph_ref[...]
    plo = plo_ref[...]

    acc_hi = jnp.zeros((nb, _CHALF), dtype=jnp.float32)
    acc_lo = jnp.zeros((nb, _CHALF), dtype=jnp.float32)
    for g in range(_G):
        vnb_g = vnb_ref[:, g * 8:(g + 1) * 8, :].reshape(nb * 8, _CHALF)
        pr_g = pf[:, g * 8:(g + 1) * 8, :].reshape(nb * 8, _H)
        w_hi = jnp.dot(pr_g, ph, preferred_element_type=jnp.float32)
        w_lo = jnp.dot(pr_g, plo, preferred_element_type=jnp.float32)
        acc_hi = acc_hi + jnp.sum(
            (w_hi * _unpack_hi(vnb_g)).reshape(nb, 8, _CHALF), axis=1)
        acc_lo = acc_lo + jnp.sum(
            (w_lo * _unpack_lo(vnb_g)).reshape(nb, 8, _CHALF), axis=1)
    hid_ref[...] = jnp.concatenate([acc_hi, acc_lo], axis=1)


def kernel(input_q, input_k, input_v, rpe_knn_embeddings, knn_idx,
           Wq, bq, Wq1, bq1, Wk, bk, Wv, bv, Wp, bp):
    xq = input_q.reshape(_N, _C)
    xk = input_k.reshape(_N, _C)
    xv = input_v.reshape(_N, _C)
    rpe = rpe_knn_embeddings.reshape(_N, _K, _C)
    idx_pad = jnp.pad(knn_idx.reshape(_N, _K).astype(jnp.int32),
                      ((0, 0), (0, _KP - _K)))
    idx_flat = idx_pad[:_MSC].reshape(1, _NIDX)

    b2 = lambda b: b.reshape(1, _C)

    q, q1, k_bf, v_bf, v_pack = pl.pallas_call(
        _proj_body,
        grid=(_N // _PROJ_NB,),
        in_specs=[
            pl.BlockSpec((_PROJ_NB, _C), lambda i: (i, 0)),
            pl.BlockSpec((_PROJ_NB, _C), lambda i: (i, 0)),
            pl.BlockSpec((_PROJ_NB, _C), lambda i: (i, 0)),
        ] + [
            spec for _ in range(4) for spec in (
                pl.BlockSpec((_C, _C), lambda i: (0, 0)),
                pl.BlockSpec((1, _C), lambda i: (0, 0)),
            )
        ],
        out_specs=[pl.BlockSpec((_PROJ_NB, _C), lambda i: (i, 0))] * 4
        + [pl.BlockSpec((_PROJ_NB, _CHALF), lambda i: (i, 0))],
        out_shape=[jax.ShapeDtypeStruct((_N, _C), jnp.float32)] * 2
        + [jax.ShapeDtypeStruct((_N, _C), jnp.bfloat16)] * 2
        + [jax.ShapeDtypeStruct((_N, _CHALF), jnp.int32)],
    )(xq, xk, xv, Wq.T, b2(bq), Wq1.T, b2(bq1), Wk.T, b2(bk), Wv.T, b2(bv))

    mesh = plsc.VectorSubcoreMesh(core_axis_name="c", subcore_axis_name="s")
    sc_gather = pl.kernel(
        _sc_gather_body,
        mesh=mesh,
        out_type=jax.ShapeDtypeStruct((_NIDX, _CHALF), jnp.int32),
        scratch_types=[],
    )
    vnb = sc_gather(v_pack, idx_flat).reshape(_MSC, _KP, _CHALF)

    pool = jnp.repeat(jnp.eye(_H, dtype=jnp.float32), _D, axis=0)
    eye_h = jnp.eye(_H, dtype=jnp.float32)
    pool_hi = jnp.repeat(eye_h[:, :_H // 2], _D, axis=1)
    pool_lo = jnp.repeat(eye_h[:, _H // 2:], _D, axis=1)

    probs_full = pl.pallas_call(
        _score_body,
        grid=(_N // _ATTN_NB,),
        in_specs=[
            pl.BlockSpec((_ATTN_NB, _C), lambda i: (i, 0)),
            pl.BlockSpec((_ATTN_NB, _C), lambda i: (i, 0)),
            pl.BlockSpec((_N, _C), lambda i: (0, 0)),
            pl.BlockSpec((_ATTN_NB, _KP), lambda i: (i, 0)),
            pl.BlockSpec((_ATTN_NB, _K, _C), lambda i: (i, 0, 0)),
            pl.BlockSpec((_C, _C), lambda i: (0, 0)),
            pl.BlockSpec((1, _C), lambda i: (0, 0)),
            pl.BlockSpec((_C, _H), lambda i: (0, 0)),
        ],
        out_specs=pl.BlockSpec((_ATTN_NB, _KP, _H), lambda i: (i, 0, 0)),
        out_shape=jax.ShapeDtypeStruct((_N, _KP, _H), jnp.float32),
    )(q, q1, k_bf, idx_pad, rpe, Wp.T.astype(jnp.bfloat16), b2(bp), pool)

    hid_a, probs_a = pl.pallas_call(
        _final_body,
        grid=(_MSC // _ATTN_NB,),
        in_specs=[
            pl.BlockSpec((_ATTN_NB, _KP, _CHALF), lambda i: (i, 0, 0)),
            pl.BlockSpec((_ATTN_NB, _KP, _H), lambda i: (i, 0, 0)),
            pl.BlockSpec((_H, _CHALF), lambda i: (0, 0)),
            pl.BlockSpec((_H, _CHALF), lambda i: (0, 0)),
        ],
        out_specs=[
            pl.BlockSpec((_ATTN_NB, _C), lambda i: (i, 0)),
            pl.BlockSpec((_ATTN_NB, _K, _H), lambda i: (i, 0, 0)),
        ],
        out_shape=[
            jax.ShapeDtypeStruct((_MSC, _C), jnp.float32),
            jax.ShapeDtypeStruct((_MSC, _K, _H), jnp.float32),
        ],
    )(vnb, probs_full[:_MSC], pool_hi, pool_lo)

    hid_b, probs_b = pl.pallas_call(
        _final_tc_body,
        grid=(_MTC // _ATTN_NB,),
        in_specs=[
            pl.BlockSpec((_N, _C), lambda i: (0, 0)),
            pl.BlockSpec((_ATTN_NB, _KP), lambda i: (i, 0)),
            pl.BlockSpec((_ATTN_NB, _KP, _H), lambda i: (i, 0, 0)),
            pl.BlockSpec((_H, _C), lambda i: (0, 0)),
        ],
        out_specs=[
            pl.BlockSpec((_ATTN_NB, _C), lambda i: (i, 0)),
            pl.BlockSpec((_ATTN_NB, _K, _H), lambda i: (i, 0, 0)),
        ],
        out_shape=[
            jax.ShapeDtypeStruct((_MTC, _C), jnp.float32),
            jax.ShapeDtypeStruct((_MTC, _K, _H), jnp.float32),
        ],
    )(v_bf, idx_pad[_MSC:], probs_full[_MSC:], pool.T)

    hidden = jnp.concatenate([hid_a, hid_b], axis=0).reshape(_B, _N, _C)
    probs_raw = jnp.concatenate([probs_a, probs_b], axis=0)
    attention_probs = probs_raw.transpose(0, 2, 1).reshape(_B, _N, _H, _K)
    return (hidden, attention_probs)
